# baseline (speedup 1.0000x reference)
import functools

import jax
import jax.numpy as jnp
from jax import lax
from jax.experimental import pallas as pl
from jax.experimental.pallas import tpu as pltpu
from jax.experimental.pallas import tpu_sc as plsc

_NODES = 4096
_N = 16777216
_H = 1.0 / (_NODES - 1)

_NC = 2
_NS = 16
_NW = _NC * _NS
_LANES = 16

_PER_TILE = _N // _NW
_CHUNK = 16384
_NCHUNK = _PER_TILE // _CHUNK
_VECS = _CHUNK // _LANES

_TPAD = _NODES + 16
_MMLEN = _NODES + 32


def _sc_kernel(x, value):
    mesh = plsc.VectorSubcoreMesh(core_axis_name="c", subcore_axis_name="s")

    @functools.partial(
        pl.kernel,
        out_type=jax.ShapeDtypeStruct((_N,), jnp.float32),
        mesh=mesh,
        compiler_params=pltpu.CompilerParams(needs_layout_passes=False),
        scratch_types=[
            pltpu.VMEM((_TPAD,), jnp.float32),
            pltpu.VMEM((_TPAD,), jnp.float32),
            pltpu.VMEM((_TPAD,), jnp.float32),
            pltpu.VMEM((_TPAD,), jnp.int32),
            pltpu.VMEM((_MMLEN,), jnp.float32),
            pltpu.VMEM((_MMLEN,), jnp.float32),
            pltpu.VMEM((_CHUNK,), jnp.float32),
            pltpu.VMEM((_CHUNK,), jnp.float32),
            pltpu.VMEM((_CHUNK,), jnp.float32),
            pltpu.VMEM((_CHUNK,), jnp.float32),
            pltpu.SemaphoreType.DMA,
            pltpu.SemaphoreType.DMA,
            pltpu.SemaphoreType.DMA,
            pltpu.SemaphoreType.DMA,
        ],
    )
    def body(x_hbm, y_hbm, out_hbm, tab0, tab1, tab1s, tab23, mm, dm,
             xin0, xin1, res0, res1, si0, si1, so0, so1):
        cid = lax.axis_index("c")
        sid = lax.axis_index("s")
        wid = sid * _NC + cid
        base = wid * _PER_TILE

        bufs = ((xin0, res0, si0, so0), (xin1, res1, si1, so1))

        def load(c, xin, si):
            return pltpu.make_async_copy(
                x_hbm.at[pl.ds(base + c * _CHUNK, _CHUNK)], xin, si)

        def store(c, res, so):
            return pltpu.make_async_copy(
                res, out_hbm.at[pl.ds(base + c * _CHUNK, _CHUNK)], so)

        load(0, xin0, si0).start()
        load(1, xin1, si1).start()

        pltpu.sync_copy(y_hbm, tab0.at[pl.ds(0, _NODES)])

        scale = jnp.float32(_NODES - 1)

        @plsc.parallel_loop(0, _NODES, step=_LANES, unroll=4)
        def _slopes(k):
            a = tab0[pl.ds(k, _LANES)]
            b = tab0[pl.ds(k + 1, _LANES)]
            mm[pl.ds(k + 2, _LANES)] = (b - a) * scale

        iota = lax.iota(jnp.int32, _LANES)
        idx_a = jnp.where(iota < 2, 2, _NODES)
        idx_b = jnp.where(iota < 2, 3, _NODES - 1)
        va = plsc.load_gather(mm, [idx_a])
        vb = plsc.load_gather(mm, [idx_b])
        ca = jnp.where((iota & 1) == 0, 2.0, 3.0).astype(jnp.float32)
        ext = ca * va - (ca - 1.0) * vb
        dest = jnp.where(
            iota == 0, 1, jnp.where(
                iota == 1, 0, jnp.where(iota == 2, _NODES + 1, _NODES + 2)))
        plsc.store_scatter(mm, [dest], ext, mask=iota < 4)

        @plsc.parallel_loop(0, _NODES + 16, step=_LANES, unroll=4)
        def _diffs(k):
            a = mm[pl.ds(k, _LANES)]
            b = mm[pl.ds(k + 1, _LANES)]
            dm[pl.ds(k, _LANES)] = jnp.abs(b - a)

        @plsc.parallel_loop(0, _NODES, step=_LANES, unroll=2)
        def _tangents(k):
            w1 = dm[pl.ds(k + 2, _LANES)]
            w2 = dm[pl.ds(k, _LANES)]
            ma = mm[pl.ds(k + 1, _LANES)]
            mb = mm[pl.ds(k + 2, _LANES)]
            denom = w1 + w2
            ok = denom > jnp.float32(1e-9)
            safe = jnp.where(ok, denom, jnp.float32(1.0))
            tv = jnp.where(ok, (w1 * ma + w2 * mb) / safe,
                           jnp.float32(0.5) * (ma + mb))
            tab1[pl.ds(k, _LANES)] = tv

        tab1[pl.ds(_NODES, _LANES)] = jnp.zeros((_LANES,), jnp.float32)

        hh = jnp.float32(_H)

        @plsc.parallel_loop(0, _NODES, step=_LANES, unroll=2)
        def _cubics(k):
            t0 = tab1[pl.ds(k, _LANES)]
            t1 = tab1[pl.ds(k + 1, _LANES)]
            mi = mm[pl.ds(k + 2, _LANES)]
            tab1s[pl.ds(k, _LANES)] = t0 * hh
            c2 = (3.0 * mi - 2.0 * t0 - t1) * hh
            c3 = (t0 + t1 - 2.0 * mi) * hh
            packed = plsc.pack(c2, c3, format=plsc.PackFormat.INTERLEAVED)
            tab23[pl.ds(k, _LANES)] = plsc.bitcast(packed, jnp.int32)

        def outer(j, _):
            for b in range(2):
                xin, res, si, so = bufs[b]
                c = 2 * j + b
                load(c, xin, si).wait()

                @pl.when(c >= 2)
                def _drain():
                    store(c - 2, res, so).wait()

                @plsc.parallel_loop(0, _CHUNK, step=_LANES, unroll=8)
                def vec(i):
                    xv = xin[pl.ds(i, _LANES)]
                    s = xv * scale
                    si_ = s.astype(jnp.int32)
                    u = s - si_.astype(jnp.float32)
                    c0 = plsc.load_gather(tab0, [si_])
                    c1 = plsc.load_gather(tab1s, [si_])
                    g = plsc.load_gather(tab23, [si_])
                    c2, c3 = plsc.unpack(
                        plsc.bitcast(g, jnp.bfloat16),
                        format=plsc.PackFormat.INTERLEAVED)
                    res[pl.ds(i, _LANES)] = c0 + u * (c1 + u * (c2 + u * c3))

                store(c, res, so).start()

                @pl.when(c + 2 < _NCHUNK)
                def _next():
                    load(c + 2, xin, si).start()

            return _

        lax.fori_loop(0, _NCHUNK // 2, outer, None)
        store(_NCHUNK - 2, res0, so0).wait()
        store(_NCHUNK - 1, res1, so1).wait()

    return body(x, value)


@jax.jit
def kernel(input, value):
    return _sc_kernel(input, value)


# speedup vs baseline: 1.0020x; 1.0020x over previous
import functools

import jax
import jax.numpy as jnp
from jax import lax
from jax.experimental import pallas as pl
from jax.experimental.pallas import tpu as pltpu
from jax.experimental.pallas import tpu_sc as plsc

_NODES = 4096
_N = 16777216
_H = 1.0 / (_NODES - 1)

_NC = 2
_NS = 16
_NW = _NC * _NS
_LANES = 16

_PER_TILE = _N // _NW
_CHUNK = 16384
_NCHUNK = _PER_TILE // _CHUNK

_TPAD = _NODES + 16
_MMLEN = _NODES + 32


def _sc_kernel(x, value):
    mesh = plsc.VectorSubcoreMesh(core_axis_name="c", subcore_axis_name="s")

    @functools.partial(
        pl.kernel,
        out_type=jax.ShapeDtypeStruct((_N,), jnp.float32),
        mesh=mesh,
        compiler_params=pltpu.CompilerParams(needs_layout_passes=False),
        scratch_types=[
            pltpu.VMEM((_TPAD,), jnp.float32),
            pltpu.VMEM((_TPAD,), jnp.float32),
            pltpu.VMEM((_TPAD,), jnp.float32),
            pltpu.VMEM((_TPAD,), jnp.int32),
            pltpu.VMEM((_MMLEN,), jnp.float32),
            pltpu.VMEM((_MMLEN,), jnp.float32),
            pltpu.VMEM((_CHUNK,), jnp.float32),
            pltpu.VMEM((_CHUNK,), jnp.float32),
            pltpu.VMEM((_CHUNK,), jnp.float32),
            pltpu.VMEM((_CHUNK,), jnp.float32),
            pltpu.SemaphoreType.DMA,
            pltpu.SemaphoreType.DMA,
            pltpu.SemaphoreType.DMA,
            pltpu.SemaphoreType.DMA,
        ],
    )
    def body(x_hbm, y_hbm, out_hbm, tab0, tab1, tab1s, tab23, mm, dm,
             xin0, xin1, res0, res1, si0, si1, so0, so1):
        cid = lax.axis_index("c")
        sid = lax.axis_index("s")
        wid = sid * _NC + cid
        base = wid * _PER_TILE

        bufs = ((xin0, res0, si0, so0), (xin1, res1, si1, so1))

        def load(c, xin, si):
            return pltpu.make_async_copy(
                x_hbm.at[pl.ds(base + c * _CHUNK, _CHUNK)], xin, si)

        def store(c, res, so):
            return pltpu.make_async_copy(
                res, out_hbm.at[pl.ds(base + c * _CHUNK, _CHUNK)], so)

        load(0, xin0, si0).start()
        load(1, xin1, si1).start()

        pltpu.sync_copy(y_hbm, tab0.at[pl.ds(0, _NODES)])

        scale = jnp.float32(_NODES - 1)

        @plsc.parallel_loop(0, _NODES, step=_LANES, unroll=4)
        def _slopes(k):
            a = tab0[pl.ds(k, _LANES)]
            b = tab0[pl.ds(k + 1, _LANES)]
            mm[pl.ds(k + 2, _LANES)] = (b - a) * scale

        iota = lax.iota(jnp.int32, _LANES)
        idx_a = jnp.where(iota < 2, 2, _NODES)
        idx_b = jnp.where(iota < 2, 3, _NODES - 1)
        va = plsc.load_gather(mm, [idx_a])
        vb = plsc.load_gather(mm, [idx_b])
        ca = jnp.where((iota & 1) == 0, 2.0, 3.0).astype(jnp.float32)
        ext = ca * va - (ca - 1.0) * vb
        dest = jnp.where(
            iota == 0, 1, jnp.where(
                iota == 1, 0, jnp.where(iota == 2, _NODES + 1, _NODES + 2)))
        plsc.store_scatter(mm, [dest], ext, mask=iota < 4)

        @plsc.parallel_loop(0, _NODES + 16, step=_LANES, unroll=4)
        def _diffs(k):
            a = mm[pl.ds(k, _LANES)]
            b = mm[pl.ds(k + 1, _LANES)]
            dm[pl.ds(k, _LANES)] = jnp.abs(b - a)

        @plsc.parallel_loop(0, _NODES, step=_LANES, unroll=2)
        def _tangents(k):
            w1 = dm[pl.ds(k + 2, _LANES)]
            w2 = dm[pl.ds(k, _LANES)]
            ma = mm[pl.ds(k + 1, _LANES)]
            mb = mm[pl.ds(k + 2, _LANES)]
            denom = w1 + w2
            ok = denom > jnp.float32(1e-9)
            safe = jnp.where(ok, denom, jnp.float32(1.0))
            tv = jnp.where(ok, (w1 * ma + w2 * mb) / safe,
                           jnp.float32(0.5) * (ma + mb))
            tab1[pl.ds(k, _LANES)] = tv

        tab1[pl.ds(_NODES, _LANES)] = jnp.zeros((_LANES,), jnp.float32)

        hh = jnp.float32(_H)

        @plsc.parallel_loop(0, _NODES, step=_LANES, unroll=2)
        def _cubics(k):
            t0 = tab1[pl.ds(k, _LANES)]
            t1 = tab1[pl.ds(k + 1, _LANES)]
            mi = mm[pl.ds(k + 2, _LANES)]
            tab1s[pl.ds(k, _LANES)] = t0 * hh
            c2 = (3.0 * mi - 2.0 * t0 - t1) * hh
            c3 = (t0 + t1 - 2.0 * mi) * hh
            packed = plsc.pack(c2, c3, format=plsc.PackFormat.INTERLEAVED)
            tab23[pl.ds(k, _LANES)] = plsc.bitcast(packed, jnp.int32)

        def outer(j, _):
            for b in range(2):
                xin, res, si, so = bufs[b]
                c = 2 * j + b
                load(c, xin, si).wait()

                @pl.when(c >= 2)
                def _drain():
                    store(c - 2, res, so).wait()

                @plsc.parallel_loop(0, _CHUNK, step=_LANES, unroll=8)
                def vec(i):
                    xv = xin[pl.ds(i, _LANES)]
                    s = xv * scale
                    si_ = s.astype(jnp.int32)
                    u = s - si_.astype(jnp.float32)
                    c0 = plsc.load_gather(tab0, [si_])
                    c1 = plsc.load_gather(tab1s, [si_])
                    g = plsc.load_gather(tab23, [si_])
                    c2, c3 = plsc.unpack(
                        plsc.bitcast(g, jnp.bfloat16),
                        format=plsc.PackFormat.INTERLEAVED)
                    res[pl.ds(i, _LANES)] = c0 + u * (c1 + u * (c2 + u * c3))

                store(c, res, so).start()

                @pl.when(c + 2 < _NCHUNK)
                def _next():
                    load(c + 2, xin, si).start()

            return _

        lax.fori_loop(0, _NCHUNK // 2, outer, None)
        store(_NCHUNK - 2, res0, so0).wait()
        store(_NCHUNK - 1, res1, so1).wait()

    return body(x, value)


@jax.jit
def kernel(input, value):
    return _sc_kernel(input, value)
